# baseline (speedup 1.0000x reference)
import functools

import jax
import jax.numpy as jnp
from jax import lax
from jax.experimental import pallas as pl
from jax.experimental.pallas import tpu as pltpu
from jax.experimental.pallas import tpu_sc as plsc

NSEG = 10000
D = 128
ROWS = 320000
NC = 2
NS = 16
NW = NC * NS
RPW = ROWS // NW
CH = 80
NCHUNK = RPW // CH
NSEG_PAD = 10240
SEG_PER_TILE = NSEG_PAD // NS
ZROWS = 16


def _sc_partial_segsum(src, idx3d):
    mesh = plsc.VectorSubcoreMesh(core_axis_name="c", subcore_axis_name="s")

    @functools.partial(
        pl.kernel,
        out_type=jax.ShapeDtypeStruct((NC, NSEG_PAD, D), jnp.float32),
        mesh=mesh,
        scratch_types=[
            pltpu.VMEM((CH, D), jnp.float32),
            pltpu.VMEM((CH, D), jnp.float32),
            pltpu.VMEM((NCHUNK, CH), jnp.int32),
            pltpu.VMEM_SHARED((NSEG_PAD, D), jnp.float32),
            pltpu.SemaphoreType.DMA,
            pltpu.SemaphoreType.DMA,
            pltpu.SemaphoreType.DMA,
            pltpu.SemaphoreType.DMA,
        ],
    )
    def k(src_hbm, idx_hbm, out_hbm, rows0, rows1, idx_v, acc_sh,
          ls0, ls1, ss0, ss1):
        c = lax.axis_index("c")
        s = lax.axis_index("s")
        wid = c * NS + s
        row0 = wid * RPW

        rows = (rows0, rows1)
        lsem = (ls0, ls1)
        ssem = (ss0, ss1)

        zeros16 = jnp.zeros((16,), jnp.float32)
        for i in range(ZROWS):
            for j in range(D // 16):
                rows0[i, pl.ds(j * 16, 16)] = zeros16
        ztpl = rows0.at[pl.ds(0, ZROWS)]
        for i in range(SEG_PER_TILE // ZROWS):
            pltpu.async_copy(
                ztpl, acc_sh.at[pl.ds(s * SEG_PER_TILE + i * ZROWS, ZROWS)],
                ls0)
        for i in range(SEG_PER_TILE // ZROWS):
            pltpu.make_async_copy(
                ztpl, acc_sh.at[pl.ds(s * SEG_PER_TILE + i * ZROWS, ZROWS)],
                ls0).wait()

        pltpu.sync_copy(idx_hbm.at[wid], idx_v)

        def load(g, b):
            base = pl.multiple_of(row0 + g * CH, CH)
            return pltpu.make_async_copy(src_hbm.at[pl.ds(base, CH)], rows[b],
                                         lsem[b])

        def scatter(g, b):
            return pltpu.make_async_copy(rows[b], acc_sh.at[idx_v.at[g]],
                                         ssem[b])

        load(0, 0).start()
        load(1, 1).start()
        plsc.subcore_barrier()

        def body(i, _):
            g = 2 * i
            load(g, 0).wait()
            scatter(g, 0).start(add=True)
            load(g + 1, 1).wait()
            scatter(g + 1, 1).start(add=True)
            scatter(g, 0).wait()
            load(g + 2, 0).start()
            scatter(g + 1, 1).wait()

            @pl.when(i < NCHUNK // 2 - 1)
            def _():
                load(g + 3, 1).start()

            return 0

        lax.fori_loop(0, NCHUNK // 2, body, 0)

        load(NCHUNK - 1, 0).wait()
        scatter(NCHUNK - 1, 0).start(add=True)
        scatter(NCHUNK - 1, 0).wait()

        plsc.subcore_barrier()
        pltpu.sync_copy(
            acc_sh.at[pl.ds(s * SEG_PER_TILE, SEG_PER_TILE)],
            out_hbm.at[c, pl.ds(s * SEG_PER_TILE, SEG_PER_TILE)],
        )

    return k(src, idx3d)


def _tc_add_partials(partials):
    def body(p_ref, o_ref):
        o_ref[...] = p_ref[0] + p_ref[1]

    blk = NSEG // 10
    return pl.pallas_call(
        body,
        out_shape=jax.ShapeDtypeStruct((NSEG, D), jnp.float32),
        grid=(NSEG // blk,),
        in_specs=[pl.BlockSpec((NC, blk, D), lambda i: (0, i, 0))],
        out_specs=pl.BlockSpec((blk, D), lambda i: (i, 0)),
    )(partials)


def kernel(src, index, dim_size):
    idx3d = index.astype(jnp.int32).reshape(NW, NCHUNK, CH)
    partials = _sc_partial_segsum(src, idx3d)
    return _tc_add_partials(partials)


# speedup vs baseline: 1.3878x; 1.3878x over previous
import functools

import jax
import jax.numpy as jnp
from jax import lax
from jax.experimental import pallas as pl
from jax.experimental.pallas import tpu as pltpu
from jax.experimental.pallas import tpu_sc as plsc

NSEG = 10000
D = 128
ROWS = 320000
NC = 2
NS = 16
NW = NC * NS
RPW = ROWS // NW
CH = 80
NCHUNK = RPW // CH
NBUF = 4
GCH = 32
NGRP = 4
NSEG_PAD = 10240
SEG_PER_TILE = NSEG_PAD // NS
ZROWS = 16


def _sc_partial_segsum(src, idx4d):
    mesh = plsc.VectorSubcoreMesh(core_axis_name="c", subcore_axis_name="s")

    @functools.partial(
        pl.kernel,
        out_type=jax.ShapeDtypeStruct((NC, NSEG_PAD, D), jnp.float32),
        mesh=mesh,
        scratch_types=[
            pltpu.VMEM((CH, D), jnp.float32),
            pltpu.VMEM((CH, D), jnp.float32),
            pltpu.VMEM((CH, D), jnp.float32),
            pltpu.VMEM((CH, D), jnp.float32),
            pltpu.VMEM((GCH, CH), jnp.int32),
            pltpu.VMEM((GCH, CH), jnp.int32),
            pltpu.VMEM_SHARED((NSEG_PAD, D), jnp.float32),
            pltpu.SemaphoreType.DMA,
            pltpu.SemaphoreType.DMA,
            pltpu.SemaphoreType.DMA,
            pltpu.SemaphoreType.DMA,
            pltpu.SemaphoreType.DMA,
            pltpu.SemaphoreType.DMA,
        ],
    )
    def k(src_hbm, idx_hbm, out_hbm, rows0, rows1, rows2, rows3,
          idxb0, idxb1, acc_sh, ls0, ls1, ls2, ls3, is0, is1):
        c = lax.axis_index("c")
        s = lax.axis_index("s")
        wid = c * NS + s
        row0 = wid * RPW

        rows = (rows0, rows1, rows2, rows3)
        lsem = (ls0, ls1, ls2, ls3)
        idxb = (idxb0, idxb1)
        isem = (is0, is1)

        zeros16 = jnp.zeros((16,), jnp.float32)
        for i in range(ZROWS):
            for j in range(D // 16):
                rows0[i, pl.ds(j * 16, 16)] = zeros16
        ztpl = rows0.at[pl.ds(0, ZROWS)]
        for i in range(SEG_PER_TILE // ZROWS):
            pltpu.async_copy(
                ztpl, acc_sh.at[pl.ds(s * SEG_PER_TILE + i * ZROWS, ZROWS)],
                ls0)
        for i in range(SEG_PER_TILE // ZROWS):
            pltpu.make_async_copy(
                ztpl, acc_sh.at[pl.ds(s * SEG_PER_TILE + i * ZROWS, ZROWS)],
                ls0).wait()

        def load(g, b):
            base = pl.multiple_of(row0 + g * CH, CH)
            return pltpu.make_async_copy(src_hbm.at[pl.ds(base, CH)], rows[b],
                                         lsem[b])

        def gload(t):
            return pltpu.make_async_copy(idx_hbm.at[wid, t], idxb[t % 2],
                                         isem[t % 2])

        def scatter(b, idx_row):
            pltpu.sync_copy(rows[b], acc_sh.at[idx_row], add=True)

        gload(0).start()
        gload(1).start()
        for b in range(NBUF):
            load(b, b).start()
        plsc.subcore_barrier()

        for t in range(NGRP):
            gload(t).wait()
            ib = idxb[t % 2]
            g0 = t * GCH
            iters = GCH // NBUF if t < NGRP - 1 else (NCHUNK - g0 - 1) // NBUF

            def body(i, _, g0=g0, ib=ib):
                g = g0 + NBUF * i
                for b in range(NBUF):
                    load(g + b, b).wait()
                    scatter(b, ib.at[NBUF * i + b])

                    @pl.when(g + NBUF + b < NCHUNK)
                    def _():
                        load(g + NBUF + b, b).start()

                return 0

            lax.fori_loop(0, iters, body, 0)
            if t + 2 < NGRP:
                gload(t + 2).start()

        tb = (NCHUNK - 1) % NBUF
        load(NCHUNK - 1, tb).wait()
        scatter(tb, idxb[(NGRP - 1) % 2].at[(NCHUNK - 1) - (NGRP - 1) * GCH])

        plsc.subcore_barrier()
        pltpu.sync_copy(
            acc_sh.at[pl.ds(s * SEG_PER_TILE, SEG_PER_TILE)],
            out_hbm.at[c, pl.ds(s * SEG_PER_TILE, SEG_PER_TILE)],
        )

    return k(src, idx4d)


def _tc_add_partials(partials):
    def body(p_ref, o_ref):
        o_ref[...] = p_ref[0] + p_ref[1]

    blk = NSEG // 10
    return pl.pallas_call(
        body,
        out_shape=jax.ShapeDtypeStruct((NSEG, D), jnp.float32),
        grid=(NSEG // blk,),
        in_specs=[pl.BlockSpec((NC, blk, D), lambda i: (0, i, 0))],
        out_specs=pl.BlockSpec((blk, D), lambda i: (i, 0)),
    )(partials)


def kernel(src, index, dim_size):
    idx = index.astype(jnp.int32).reshape(NW, NCHUNK, CH)
    idx4d = jnp.pad(idx, ((0, 0), (0, NGRP * GCH - NCHUNK), (0, 0)))
    idx4d = idx4d.reshape(NW, NGRP, GCH, CH)
    partials = _sc_partial_segsum(src, idx4d)
    return _tc_add_partials(partials)


# speedup vs baseline: 1.4676x; 1.0575x over previous
import functools

import jax
import jax.numpy as jnp
from jax import lax
from jax.experimental import pallas as pl
from jax.experimental.pallas import tpu as pltpu
from jax.experimental.pallas import tpu_sc as plsc

NSEG = 10000
D = 128
ROWS = 320000
NC = 2
NS = 16
NW = NC * NS
RPW = ROWS // NW
CH = 80
NCHUNK = RPW // CH
NBUF = 4
GCH = 32
NGRP = 4
NSEG_PAD = 10240
SEG_PER_TILE = NSEG_PAD // NS
ZROWS = 16


def _sc_partial_segsum(src, idx4d):
    mesh = plsc.VectorSubcoreMesh(core_axis_name="c", subcore_axis_name="s")

    @functools.partial(
        pl.kernel,
        out_type=jax.ShapeDtypeStruct((NC, NSEG_PAD, D), jnp.float32),
        mesh=mesh,
        scratch_types=[
            pltpu.VMEM((CH, D), jnp.float32),
            pltpu.VMEM((CH, D), jnp.float32),
            pltpu.VMEM((CH, D), jnp.float32),
            pltpu.VMEM((CH, D), jnp.float32),
            pltpu.VMEM((GCH, CH), jnp.int32),
            pltpu.VMEM((GCH, CH), jnp.int32),
            pltpu.VMEM_SHARED((NSEG_PAD, D), jnp.float32),
            pltpu.SemaphoreType.DMA,
            pltpu.SemaphoreType.DMA,
            pltpu.SemaphoreType.DMA,
            pltpu.SemaphoreType.DMA,
            pltpu.SemaphoreType.DMA,
            pltpu.SemaphoreType.DMA,
        ],
    )
    def k(src_hbm, idx_hbm, out_hbm, rows0, rows1, rows2, rows3,
          idxb0, idxb1, acc_sh, ls0, ls1, ls2, ls3, is0, is1):
        c = lax.axis_index("c")
        s = lax.axis_index("s")
        wid = c * NS + s
        row0 = wid * RPW

        rows = (rows0, rows1, rows2, rows3)
        lsem = (ls0, ls1, ls2, ls3)
        idxb = (idxb0, idxb1)
        isem = (is0, is1)

        zeros16 = jnp.zeros((16,), jnp.float32)
        for i in range(ZROWS):
            for j in range(D // 16):
                rows0[i, pl.ds(j * 16, 16)] = zeros16
        ztpl = rows0.at[pl.ds(0, ZROWS)]
        for i in range(SEG_PER_TILE // ZROWS):
            pltpu.async_copy(
                ztpl, acc_sh.at[pl.ds(s * SEG_PER_TILE + i * ZROWS, ZROWS)],
                ls0)

        def load(g, b):
            base = pl.multiple_of(row0 + g * CH, CH)
            return pltpu.make_async_copy(src_hbm.at[pl.ds(base, CH)], rows[b],
                                         lsem[b])

        def gload(t):
            return pltpu.make_async_copy(idx_hbm.at[wid, t], idxb[t % 2],
                                         isem[t % 2])

        def scatter(b, idx_row):
            pltpu.sync_copy(rows[b], acc_sh.at[idx_row], add=True)

        gload(0).start()
        gload(1).start()
        for b in range(1, NBUF):
            load(b, b).start()
        for i in range(SEG_PER_TILE // ZROWS):
            pltpu.make_async_copy(
                ztpl, acc_sh.at[pl.ds(s * SEG_PER_TILE + i * ZROWS, ZROWS)],
                ls0).wait()
        load(0, 0).start()
        plsc.subcore_barrier()

        for t in range(NGRP):
            gload(t).wait()
            ib = idxb[t % 2]
            g0 = t * GCH
            iters = GCH // NBUF if t < NGRP - 1 else (NCHUNK - g0 - 1) // NBUF

            def body(i, _, g0=g0, ib=ib):
                g = g0 + NBUF * i
                for b in range(NBUF):
                    load(g + b, b).wait()
                    scatter(b, ib.at[NBUF * i + b])

                    @pl.when(g + NBUF + b < NCHUNK)
                    def _():
                        load(g + NBUF + b, b).start()

                return 0

            lax.fori_loop(0, iters, body, 0)
            if t + 2 < NGRP:
                gload(t + 2).start()

        tb = (NCHUNK - 1) % NBUF
        load(NCHUNK - 1, tb).wait()
        scatter(tb, idxb[(NGRP - 1) % 2].at[(NCHUNK - 1) - (NGRP - 1) * GCH])

        plsc.subcore_barrier()
        pltpu.sync_copy(
            acc_sh.at[pl.ds(s * SEG_PER_TILE, SEG_PER_TILE)],
            out_hbm.at[c, pl.ds(s * SEG_PER_TILE, SEG_PER_TILE)],
        )

    return k(src, idx4d)


def _tc_add_partials(partials):
    def body(p_ref, o_ref):
        o_ref[...] = p_ref[0] + p_ref[1]

    blk = NSEG // 2
    return pl.pallas_call(
        body,
        out_shape=jax.ShapeDtypeStruct((NSEG, D), jnp.float32),
        grid=(NSEG // blk,),
        in_specs=[pl.BlockSpec((NC, blk, D), lambda i: (0, i, 0))],
        out_specs=pl.BlockSpec((blk, D), lambda i: (i, 0)),
    )(partials)


def kernel(src, index, dim_size):
    idx = index.astype(jnp.int32).reshape(NW, NCHUNK, CH)
    idx4d = jnp.pad(idx, ((0, 0), (0, NGRP * GCH - NCHUNK), (0, 0)))
    idx4d = idx4d.reshape(NW, NGRP, GCH, CH)
    partials = _sc_partial_segsum(src, idx4d)
    return _tc_add_partials(partials)
